# initial kernel scaffold (unmeasured)
import jax
import jax.numpy as jnp
from jax import lax
from jax.experimental import pallas as pl
from jax.experimental.pallas import tpu as pltpu

N_DEV = 4


def kernel(A, B):
    m, _ = A.shape
    _, n = B.shape
    mc = m // N_DEV

    def body(a_ref, b_ref, out_ref, send_ref, rs_recv, ag_recv,
             rs_send_sems, rs_recv_sems, ag_send_sems, ag_recv_sems):
        my = lax.axis_index("i")
        left = (my - 1) % N_DEV
        right = (my + 1) % N_DEV

        barrier = pltpu.get_barrier_semaphore()
        for nbr in (left, right):
            pl.semaphore_signal(barrier, inc=1, device_id=(nbr,),
                                device_id_type=pl.DeviceIdType.MESH)
        pl.semaphore_wait(barrier, 2)

        def partial(c):
            return jnp.dot(a_ref[pl.ds(c * mc, mc), :], b_ref[:, :],
                           preferred_element_type=jnp.float32)

        send_ref[:, :] = partial((my - 1) % N_DEV)
        for s in range(N_DEV - 1):
            rdma = pltpu.make_async_remote_copy(
                src_ref=send_ref,
                dst_ref=rs_recv.at[s],
                send_sem=rs_send_sems.at[s],
                recv_sem=rs_recv_sems.at[s],
                device_id=(right,),
                device_id_type=pl.DeviceIdType.MESH,
            )
            rdma.start()
            rdma.wait()
            c = (my - 2 - s) % N_DEV
            acc = rs_recv[s] + partial(c)
            if s < N_DEV - 2:
                send_ref[:, :] = acc
            else:
                silu = acc * (1.0 / (1.0 + jnp.exp(-acc)))
                out_ref[pl.ds(my * mc, mc), :] = silu
                send_ref[:, :] = silu

        for h in range(N_DEV - 1):
            src = send_ref if h == 0 else ag_recv.at[h - 1]
            rdma = pltpu.make_async_remote_copy(
                src_ref=src,
                dst_ref=ag_recv.at[h],
                send_sem=ag_send_sems.at[h],
                recv_sem=ag_recv_sems.at[h],
                device_id=(right,),
                device_id_type=pl.DeviceIdType.MESH,
            )
            rdma.start()
            rdma.wait()
            origin = (my - 1 - h) % N_DEV
            out_ref[pl.ds(origin * mc, mc), :] = ag_recv[h]

    return pl.pallas_call(
        body,
        out_shape=jax.ShapeDtypeStruct((m, n), jnp.float32),
        in_specs=[
            pl.BlockSpec(memory_space=pltpu.VMEM),
            pl.BlockSpec(memory_space=pltpu.VMEM),
        ],
        out_specs=pl.BlockSpec(memory_space=pltpu.VMEM),
        scratch_shapes=[
            pltpu.VMEM((mc, n), jnp.float32),
            pltpu.VMEM((3, mc, n), jnp.float32),
            pltpu.VMEM((3, mc, n), jnp.float32),
            pltpu.SemaphoreType.DMA((3,)),
            pltpu.SemaphoreType.DMA((3,)),
            pltpu.SemaphoreType.DMA((3,)),
            pltpu.SemaphoreType.DMA((3,)),
        ],
        compiler_params=pltpu.CompilerParams(collective_id=0),
    )(A, B)


# baseline (device time: 318713 ns/iter reference)
import jax
import jax.numpy as jnp
from jax import lax
from jax.experimental import pallas as pl
from jax.experimental.pallas import tpu as pltpu

N_DEV = 4


def kernel(A, B):
    m, _ = A.shape
    _, n = B.shape
    mc = m // N_DEV

    def body(a_ref, b_ref, out_ref, send_ref, rs_recv,
             rs_send_sems, rs_recv_sems, ag_send_sems, ag_recv_sems):
        my = lax.axis_index("i")
        left = (my - 1) % N_DEV
        right = (my + 1) % N_DEV

        barrier = pltpu.get_barrier_semaphore()
        for nbr in (left, right):
            pl.semaphore_signal(barrier, inc=1, device_id=(nbr,),
                                device_id_type=pl.DeviceIdType.MESH)
        pl.semaphore_wait(barrier, 2)

        def partial(c):
            return jnp.dot(a_ref[pl.ds(c * mc, mc), :], b_ref[:, :],
                           preferred_element_type=jnp.float32)

        send_ref[:, :] = partial((my - 1) % N_DEV)
        for s in range(N_DEV - 1):
            rdma = pltpu.make_async_remote_copy(
                src_ref=send_ref,
                dst_ref=rs_recv.at[s % 2],
                send_sem=rs_send_sems.at[s],
                recv_sem=rs_recv_sems.at[s],
                device_id=(right,),
                device_id_type=pl.DeviceIdType.MESH,
            )
            rdma.start()
            rdma.wait()
            c = (my - 2 - s) % N_DEV
            acc = rs_recv[s % 2] + partial(c)
            if s < N_DEV - 2:
                send_ref[:, :] = acc
            else:
                out_ref[pl.ds(my * mc, mc), :] = acc * (
                    1.0 / (1.0 + jnp.exp(-acc)))

        for h in range(N_DEV - 1):
            stripe = (my - h) % N_DEV
            rdma = pltpu.make_async_remote_copy(
                src_ref=out_ref.at[pl.ds(stripe * mc, mc), :],
                dst_ref=out_ref.at[pl.ds(stripe * mc, mc), :],
                send_sem=ag_send_sems.at[h],
                recv_sem=ag_recv_sems.at[h],
                device_id=(right,),
                device_id_type=pl.DeviceIdType.MESH,
            )
            rdma.start()
            rdma.wait()

    return pl.pallas_call(
        body,
        out_shape=jax.ShapeDtypeStruct((m, n), jnp.float32),
        in_specs=[
            pl.BlockSpec(memory_space=pltpu.VMEM),
            pl.BlockSpec(memory_space=pltpu.VMEM),
        ],
        out_specs=pl.BlockSpec(memory_space=pltpu.VMEM),
        scratch_shapes=[
            pltpu.VMEM((mc, n), jnp.float32),
            pltpu.VMEM((2, mc, n), jnp.float32),
            pltpu.SemaphoreType.DMA((3,)),
            pltpu.SemaphoreType.DMA((3,)),
            pltpu.SemaphoreType.DMA((3,)),
            pltpu.SemaphoreType.DMA((3,)),
        ],
        compiler_params=pltpu.CompilerParams(collective_id=0),
    )(A, B)


# device time: 177702 ns/iter; 1.7935x vs baseline; 1.7935x over previous
import jax
import jax.numpy as jnp
from jax import lax
from jax.experimental import pallas as pl
from jax.experimental.pallas import tpu as pltpu

N_DEV = 4


def kernel(A, B):
    m, _ = A.shape
    _, n = B.shape
    mc = m // N_DEV
    nh = n // 2

    def body(a_ref, b_ref, out_ref, send_r, send_l, recv_r, recv_l,
             rs_ssem_r, rs_rsem_r, rs_ssem_l, rs_rsem_l,
             ag_ssem_r, ag_rsem_r, ag_ssem_l, ag_rsem_l):
        my = lax.axis_index("i")
        left = (my - 1) % N_DEV
        right = (my + 1) % N_DEV

        barrier = pltpu.get_barrier_semaphore()
        for nbr in (left, right):
            pl.semaphore_signal(barrier, inc=1, device_id=(nbr,),
                                device_id_type=pl.DeviceIdType.MESH)
        pl.semaphore_wait(barrier, 2)

        def partial_l(c):
            return jnp.dot(a_ref[pl.ds(c * mc, mc), :], b_ref[:, :nh],
                           preferred_element_type=jnp.float32)

        def partial_r(c):
            return jnp.dot(a_ref[pl.ds(c * mc, mc), :], b_ref[:, nh:],
                           preferred_element_type=jnp.float32)

        send_r[:, :] = partial_l((my - 1) % N_DEV)
        send_l[:, :] = partial_r((my + 1) % N_DEV)
        for s in range(N_DEV - 1):
            rdma_r = pltpu.make_async_remote_copy(
                src_ref=send_r, dst_ref=recv_r.at[s % 2],
                send_sem=rs_ssem_r.at[s], recv_sem=rs_rsem_r.at[s],
                device_id=(right,), device_id_type=pl.DeviceIdType.MESH,
            )
            rdma_l = pltpu.make_async_remote_copy(
                src_ref=send_l, dst_ref=recv_l.at[s % 2],
                send_sem=rs_ssem_l.at[s], recv_sem=rs_rsem_l.at[s],
                device_id=(left,), device_id_type=pl.DeviceIdType.MESH,
            )
            rdma_r.start()
            rdma_l.start()
            pr = partial_l((my - 2 - s) % N_DEV)
            pll = partial_r((my + 2 + s) % N_DEV)
            rdma_r.wait()
            rdma_l.wait()
            if s < N_DEV - 2:
                send_r[:, :] = recv_r[s % 2] + pr
                send_l[:, :] = recv_l[s % 2] + pll
            else:
                acc_r = recv_r[s % 2] + pr
                acc_l = recv_l[s % 2] + pll
                out_ref[pl.ds(my * mc, mc), :nh] = acc_r * (
                    1.0 / (1.0 + jnp.exp(-acc_r)))
                out_ref[pl.ds(my * mc, mc), nh:] = acc_l * (
                    1.0 / (1.0 + jnp.exp(-acc_l)))

        for h in range(N_DEV - 1):
            st_r = (my - h) % N_DEV
            st_l = (my + h) % N_DEV
            rdma_r = pltpu.make_async_remote_copy(
                src_ref=out_ref.at[pl.ds(st_r * mc, mc), pl.ds(0, nh)],
                dst_ref=out_ref.at[pl.ds(st_r * mc, mc), pl.ds(0, nh)],
                send_sem=ag_ssem_r.at[h], recv_sem=ag_rsem_r.at[h],
                device_id=(right,), device_id_type=pl.DeviceIdType.MESH,
            )
            rdma_l = pltpu.make_async_remote_copy(
                src_ref=out_ref.at[pl.ds(st_l * mc, mc), pl.ds(nh, nh)],
                dst_ref=out_ref.at[pl.ds(st_l * mc, mc), pl.ds(nh, nh)],
                send_sem=ag_ssem_l.at[h], recv_sem=ag_rsem_l.at[h],
                device_id=(left,), device_id_type=pl.DeviceIdType.MESH,
            )
            rdma_r.start()
            rdma_l.start()
            rdma_r.wait()
            rdma_l.wait()

    return pl.pallas_call(
        body,
        out_shape=jax.ShapeDtypeStruct((m, n), jnp.float32),
        in_specs=[
            pl.BlockSpec(memory_space=pltpu.VMEM),
            pl.BlockSpec(memory_space=pltpu.VMEM),
        ],
        out_specs=pl.BlockSpec(memory_space=pltpu.VMEM),
        scratch_shapes=[
            pltpu.VMEM((mc, nh), jnp.float32),
            pltpu.VMEM((mc, nh), jnp.float32),
            pltpu.VMEM((2, mc, nh), jnp.float32),
            pltpu.VMEM((2, mc, nh), jnp.float32),
            pltpu.SemaphoreType.DMA((3,)),
            pltpu.SemaphoreType.DMA((3,)),
            pltpu.SemaphoreType.DMA((3,)),
            pltpu.SemaphoreType.DMA((3,)),
            pltpu.SemaphoreType.DMA((3,)),
            pltpu.SemaphoreType.DMA((3,)),
            pltpu.SemaphoreType.DMA((3,)),
            pltpu.SemaphoreType.DMA((3,)),
        ],
        compiler_params=pltpu.CompilerParams(collective_id=0),
    )(A, B)


# device time: 166982 ns/iter; 1.9087x vs baseline; 1.0642x over previous
import jax
import jax.numpy as jnp
from jax import lax
from jax.experimental import pallas as pl
from jax.experimental.pallas import tpu as pltpu

N_DEV = 4
R, L = 0, 1


def kernel(A, B):
    m, _ = A.shape
    _, n = B.shape
    mc = m // N_DEV
    nq = n // 4

    def body(a_ref, b_ref, out_ref, sbuf_r, sbuf_l, rbuf_r, rbuf_l,
             rs_ss_r, rs_rs_r, rs_ss_l, rs_rs_l,
             ag_ss_r, ag_rs_r, ag_ss_l, ag_rs_l):
        my = lax.axis_index("i")
        left = (my - 1) % N_DEV
        right = (my + 1) % N_DEV

        barrier = pltpu.get_barrier_semaphore()
        for nbr in (left, right):
            pl.semaphore_signal(barrier, inc=1, device_id=(nbr,),
                                device_id_type=pl.DeviceIdType.MESH)
        pl.semaphore_wait(barrier, 2)

        def partial(c, q):
            return jnp.dot(a_ref[pl.ds(c * mc, mc), :],
                           b_ref[:, q * nq:(q + 1) * nq],
                           preferred_element_type=jnp.float32)

        def rs_rdma(d, b, s):
            sbuf, rbuf = (sbuf_r, rbuf_r) if d == R else (sbuf_l, rbuf_l)
            ss, rs = (rs_ss_r, rs_rs_r) if d == R else (rs_ss_l, rs_rs_l)
            return pltpu.make_async_remote_copy(
                src_ref=sbuf.at[b],
                dst_ref=rbuf.at[b, s % 2],
                send_sem=ss.at[b, s],
                recv_sem=rs.at[b, s],
                device_id=(right if d == R else left,),
                device_id_type=pl.DeviceIdType.MESH,
            )

        def ag_rdma(d, b, h):
            stripe = (my - h) % N_DEV if d == R else (my + h) % N_DEV
            q = b if d == R else 2 + b
            region = out_ref.at[pl.ds(stripe * mc, mc), pl.ds(q * nq, nq)]
            ss, rs = (ag_ss_r, ag_rs_r) if d == R else (ag_ss_l, ag_rs_l)
            return pltpu.make_async_remote_copy(
                src_ref=region, dst_ref=region,
                send_sem=ss.at[b, h],
                recv_sem=rs.at[b, h],
                device_id=(right if d == R else left,),
                device_id_type=pl.DeviceIdType.MESH,
            )

        first = (my - 1) % N_DEV
        firstl = (my + 1) % N_DEV

        sbuf_r[0] = partial(first, 0)
        rs_rdma(R, 0, 0).start()
        sbuf_l[0] = partial(firstl, 2)
        rs_rdma(L, 0, 0).start()
        sbuf_r[1] = partial(first, 1)
        rs_rdma(R, 1, 0).start()
        sbuf_l[1] = partial(firstl, 3)
        rs_rdma(L, 1, 0).start()

        for s in (1, 2):
            cr = (my - 1 - s) % N_DEV
            cl = (my + 1 + s) % N_DEV
            for b in (0, 1):
                pr = partial(cr, b)
                rs_rdma(R, b, s - 1).wait_send()
                sbuf_r[b] = pr
                rs_rdma(R, b, s - 1).wait_recv()
                sbuf_r[b] += rbuf_r[b, (s - 1) % 2]
                rs_rdma(R, b, s).start()

                pll = partial(cl, 2 + b)
                rs_rdma(L, b, s - 1).wait_send()
                sbuf_l[b] = pll
                rs_rdma(L, b, s - 1).wait_recv()
                sbuf_l[b] += rbuf_l[b, (s - 1) % 2]
                rs_rdma(L, b, s).start()

        for b in (0, 1):
            pr = partial(my, b)
            rs_rdma(R, b, 2).wait_recv()
            acc = rbuf_r[b, 0] + pr
            out_ref[pl.ds(my * mc, mc), b * nq:(b + 1) * nq] = acc * (
                1.0 / (1.0 + jnp.exp(-acc)))
            ag_rdma(R, b, 0).start()

            pll = partial(my, 2 + b)
            rs_rdma(L, b, 2).wait_recv()
            acc = rbuf_l[b, 0] + pll
            out_ref[pl.ds(my * mc, mc), (2 + b) * nq:(3 + b) * nq] = acc * (
                1.0 / (1.0 + jnp.exp(-acc)))
            ag_rdma(L, b, 0).start()

        for h in (1, 2):
            for b in (0, 1):
                ag_rdma(R, b, h - 1).wait_recv()
                ag_rdma(R, b, h).start()
                ag_rdma(L, b, h - 1).wait_recv()
                ag_rdma(L, b, h).start()

        for b in (0, 1):
            ag_rdma(R, b, 2).wait_recv()
            ag_rdma(L, b, 2).wait_recv()
            rs_rdma(R, b, 2).wait_send()
            rs_rdma(L, b, 2).wait_send()
            for h in (0, 1, 2):
                ag_rdma(R, b, h).wait_send()
                ag_rdma(L, b, h).wait_send()

    return pl.pallas_call(
        body,
        out_shape=jax.ShapeDtypeStruct((m, n), jnp.float32),
        in_specs=[
            pl.BlockSpec(memory_space=pltpu.VMEM),
            pl.BlockSpec(memory_space=pltpu.VMEM),
        ],
        out_specs=pl.BlockSpec(memory_space=pltpu.VMEM),
        scratch_shapes=[
            pltpu.VMEM((2, mc, nq), jnp.float32),
            pltpu.VMEM((2, mc, nq), jnp.float32),
            pltpu.VMEM((2, 2, mc, nq), jnp.float32),
            pltpu.VMEM((2, 2, mc, nq), jnp.float32),
            pltpu.SemaphoreType.DMA((2, 3)),
            pltpu.SemaphoreType.DMA((2, 3)),
            pltpu.SemaphoreType.DMA((2, 3)),
            pltpu.SemaphoreType.DMA((2, 3)),
            pltpu.SemaphoreType.DMA((2, 3)),
            pltpu.SemaphoreType.DMA((2, 3)),
            pltpu.SemaphoreType.DMA((2, 3)),
            pltpu.SemaphoreType.DMA((2, 3)),
        ],
        compiler_params=pltpu.CompilerParams(collective_id=0),
    )(A, B)
